# baseline (speedup 1.0000x reference)
import functools

import jax
import jax.numpy as jnp
from jax import lax
from jax.experimental import pallas as pl
from jax.experimental.pallas import tpu as pltpu
from jax.experimental.pallas import tpu_sc as plsc

SEQ = 200
BATCH = 1024
DIM = 64
N = SEQ * BATCH
NC = 2
NS = 16
NW = NC * NS
ROWS_W = N // NW
G = 128
NG = ROWS_W // G
NBUF = 4
OB = 4
NITER = NG // NBUF
LANES = 16
TBL_R = 500000


def _emb_body(tbl_hbm, idx_hbm, out_hbm, idx_v, qstage, grow, obuf, gsem, ssem):
    wid = lax.axis_index("s") * NC + lax.axis_index("c")
    base = wid * ROWS_W

    pltpu.sync_copy(idx_hbm.at[pl.ds(base * 1, ROWS_W)], idx_v)

    def stage_q(g, slot):
        for k in range(G // LANES):
            v16 = idx_v[pl.ds(g * G + k * LANES, LANES)]
            qstage[slot, pl.ds(k * LANES, LANES)] = lax.shift_right_logical(v16, 1)

    def gather_desc(slot):
        return pltpu.make_async_copy(
            tbl_hbm.at[qstage.at[slot]], grow.at[slot], gsem.at[slot])

    def store_desc(g, slot):
        n0 = (base + g * G)
        l = n0 // BATCH
        b0 = n0 % BATCH
        return pltpu.make_async_copy(
            obuf.at[slot], out_hbm.at[l, :, pl.ds(b0, G)], ssem.at[slot])

    def compute(g, gslot, oslot):
        src = grow.at[gslot]
        for bv in range(G // LANES):
            bbase = bv * LANES
            idx16 = idx_v[pl.ds(g * G + bbase, LANES)]
            h16 = lax.bitwise_and(idx16, 1)
            rows16 = lax.iota(jnp.int32, LANES) + bbase
            col0 = h16 * DIM
            scale = jnp.where(idx16 == 0, jnp.float32(0), jnp.float32(1))

            @pl.loop(0, DIM, unroll=4)
            def _per_dim(d):
                cols = col0 + d
                vals = plsc.load_gather(src, [rows16, cols])
                obuf[oslot, d, pl.ds(bbase, LANES)] = vals * scale

    def visit(g, gslot, oslot):
        gather_desc(gslot).wait()
        compute(g, gslot, oslot)
        store_desc(g, oslot).start()
        p = g + NBUF
        if isinstance(p, int) and p >= NG:
            return
        def _do():
            stage_q(p, gslot)
            gather_desc(gslot).start()
        if isinstance(p, int):
            _do()
        else:
            pl.when(p < NG)(_do)

    for b in range(NBUF):
        stage_q(b, b)
        gather_desc(b).start()

    @pl.loop(0, NITER)
    def _outer(i):
        for b in range(NBUF):
            g = i * NBUF + b
            def _drain():
                store_desc(g - OB, b).wait()
            pl.when(i >= 1)(_drain)
            visit(g, b, b)

    for g in range(NITER * NBUF, NG):
        b = g % NBUF
        store_desc(g - OB, b).wait()
        visit(g, b, b)

    for g in range(NG - OB, NG):
        store_desc(g, g % NBUF).wait()


_emb_lookup = functools.partial(
    pl.kernel,
    out_type=jax.ShapeDtypeStruct((SEQ, DIM, BATCH), jnp.float32),
    mesh=plsc.VectorSubcoreMesh(
        core_axis_name="c", subcore_axis_name="s",
        num_cores=NC, num_subcores=NS),
    scratch_types=[
        pltpu.VMEM((ROWS_W,), jnp.int32),
        pltpu.VMEM((NBUF, G), jnp.int32),
        pltpu.VMEM((NBUF, G, 128), jnp.float32),
        pltpu.VMEM((OB, DIM, G), jnp.float32),
        pltpu.SemaphoreType.DMA((NBUF,)),
        pltpu.SemaphoreType.DMA((OB,)),
    ],
    compiler_params=pltpu.CompilerParams(
        needs_layout_passes=False, use_tc_tiling_on_sc=True),
)(_emb_body)


@jax.jit
def kernel(src_input, table):
    idx = src_input.reshape(N).astype(jnp.int32)
    tbl2 = table.reshape(TBL_R, 128)
    out = _emb_lookup(tbl2, idx)
    return jnp.transpose(out, (0, 2, 1))


# speedup vs baseline: 1.2501x; 1.2501x over previous
import functools

import jax
import jax.numpy as jnp
from jax import lax
from jax.experimental import pallas as pl
from jax.experimental.pallas import tpu as pltpu
from jax.experimental.pallas import tpu_sc as plsc

SEQ = 200
BATCH = 1024
DIM = 64
N = SEQ * BATCH
NC = 2
NS = 16
NW = NC * NS
ROWS_W = N // NW
G = 128
NG = ROWS_W // G
NBUF = 5
NITER = NG // NBUF
LANES = 16


def _emb_body(table_hbm, idx_hbm, out_hbm, idx_v, rows_v, gsem, ssem):
    wid = lax.axis_index("s") * NC + lax.axis_index("c")
    base = wid * ROWS_W

    pltpu.sync_copy(idx_hbm.at[wid], idx_v)

    def gather_desc(g, slot):
        return pltpu.make_async_copy(
            table_hbm.at[idx_v.at[g]], rows_v.at[slot], gsem.at[slot])

    def store_desc(g, slot):
        n0 = base + g * G
        l = n0 // BATCH
        b0 = n0 % BATCH
        return pltpu.make_async_copy(
            rows_v.at[slot], out_hbm.at[l, pl.ds(b0, G), :], ssem.at[slot])

    for b in range(NBUF - 1):
        gather_desc(b, b).start()

    @pl.loop(0, NITER)
    def _outer(i):
        for b in range(NBUF):
            h = i * NBUF + b
            slot = b
            gather_desc(h, slot).wait()

            cnt_vec = jnp.zeros((LANES,), jnp.int32)
            for k in range(G // LANES):
                v16 = idx_v[h, pl.ds(k * LANES, LANES)]
                cnt_vec = cnt_vec + jnp.where(v16 == 0, 1, 0).astype(jnp.int32)
            cnt = jnp.sum(cnt_vec)

            @pl.when(cnt > 0)
            def _fix():
                @pl.loop(0, G // LANES)
                def _per16(k):
                    v16 = idx_v[h, pl.ds(k * LANES, LANES)]
                    scale = jnp.where(v16 == 0, jnp.float32(0), jnp.float32(1))

                    @pl.loop(0, LANES)
                    def _per_row(r):
                        lane = jnp.broadcast_to(r, (LANES,)).astype(jnp.int32)
                        srow = lax.gather(
                            scale, lane[:, None],
                            lax.GatherDimensionNumbers(
                                offset_dims=(), collapsed_slice_dims=(0,),
                                start_index_map=(0,)),
                            slice_sizes=(1,),
                            mode=lax.GatherScatterMode.PROMISE_IN_BOUNDS)
                        row = k * LANES + r
                        for c in range(DIM // LANES):
                            sl = pl.ds(c * LANES, LANES)
                            rows_v[slot, row, sl] = rows_v[slot, row, sl] * srow

            store_desc(h, slot).start()

            p = h + NBUF - 1
            slot_p = (b - 1) % NBUF
            if b == 0:
                @pl.when(p < NG)
                def _refill0():
                    @pl.when(i >= 1)
                    def _drain_prev():
                        store_desc(h - 1, slot_p).wait()
                    gather_desc(p, slot_p).start()
            else:
                @pl.when(p < NG)
                def _refill():
                    store_desc(h - 1, slot_p).wait()
                    gather_desc(p, slot_p).start()

    for b in range(NBUF):
        store_desc(NG - NBUF + b, b).wait()


_emb_lookup = functools.partial(
    pl.kernel,
    out_type=jax.ShapeDtypeStruct((SEQ, BATCH, DIM), jnp.float32),
    mesh=plsc.VectorSubcoreMesh(
        core_axis_name="c", subcore_axis_name="s",
        num_cores=NC, num_subcores=NS),
    scratch_types=[
        pltpu.VMEM((NG, G), jnp.int32),
        pltpu.VMEM((NBUF, G, DIM), jnp.float32),
        pltpu.SemaphoreType.DMA((NBUF,)),
        pltpu.SemaphoreType.DMA((NBUF,)),
    ],
    compiler_params=pltpu.CompilerParams(
        needs_layout_passes=False, use_tc_tiling_on_sc=False),
)(_emb_body)


@jax.jit
def kernel(src_input, table):
    idx = src_input.reshape(N).astype(jnp.int32).reshape(NW, NG, G)
    return _emb_lookup(table, idx)


# speedup vs baseline: 1.9064x; 1.5250x over previous
import functools

import jax
import jax.numpy as jnp
from jax import lax
from jax.experimental import pallas as pl
from jax.experimental.pallas import tpu as pltpu
from jax.experimental.pallas import tpu_sc as plsc

SEQ = 200
BATCH = 1024
DIM = 64
N = SEQ * BATCH
NC = 2
NS = 16
NW = NC * NS
ROWS_W = N // NW
G = 128
NG = ROWS_W // G
NBUF = 5
NITER = NG // NBUF
LANES = 16


def _emb_body(table_hbm, idx_hbm, out_hbm, idx_v, rows_v, idx_s, gsem, ssem):
    wid = lax.axis_index("s") * NC + lax.axis_index("c")
    base = wid * ROWS_W

    pltpu.sync_copy(idx_hbm.at[wid], idx_v)

    def start_gathers(g, slot):
        @pl.loop(0, G // LANES)
        def _k(k):
            v16 = idx_v[g, pl.ds(k * LANES, LANES)]
            for j in range(LANES):
                ridx = v16[j]
                pltpu.async_copy(
                    table_hbm.at[pl.ds(ridx, 1), :],
                    rows_v.at[slot, pl.ds(k * LANES + j, 1), :],
                    gsem.at[slot])

    def gather_drain(slot):
        return pltpu.make_async_copy(
            table_hbm.at[pl.ds(0, G), :], rows_v.at[slot], gsem.at[slot])

    def store_desc(g, slot):
        n0 = base + g * G
        l = n0 // BATCH
        b0 = n0 % BATCH
        return pltpu.make_async_copy(
            rows_v.at[slot], out_hbm.at[l, pl.ds(b0, G), :], ssem.at[slot])

    for b in range(NBUF - 1):
        start_gathers(b, b)

    @pl.loop(0, NITER)
    def _outer(i):
        for b in range(NBUF):
            h = i * NBUF + b
            slot = b
            gather_drain(slot).wait()

            cnt_vec = jnp.zeros((LANES,), jnp.int32)
            for k in range(G // LANES):
                v16 = idx_v[h, pl.ds(k * LANES, LANES)]
                cnt_vec = cnt_vec + jnp.where(v16 == 0, 1, 0).astype(jnp.int32)
            cnt = jnp.sum(cnt_vec)

            @pl.when(cnt > 0)
            def _fix():
                @pl.loop(0, G // LANES)
                def _per16(k):
                    v16 = idx_v[h, pl.ds(k * LANES, LANES)]
                    scale = jnp.where(v16 == 0, jnp.float32(0), jnp.float32(1))

                    @pl.loop(0, LANES)
                    def _per_row(r):
                        lane = jnp.broadcast_to(r, (LANES,)).astype(jnp.int32)
                        srow = lax.gather(
                            scale, lane[:, None],
                            lax.GatherDimensionNumbers(
                                offset_dims=(), collapsed_slice_dims=(0,),
                                start_index_map=(0,)),
                            slice_sizes=(1,),
                            mode=lax.GatherScatterMode.PROMISE_IN_BOUNDS)
                        row = k * LANES + r
                        for c in range(DIM // LANES):
                            sl = pl.ds(c * LANES, LANES)
                            rows_v[slot, row, sl] = rows_v[slot, row, sl] * srow

            store_desc(h, slot).start()

            p = h + NBUF - 1
            slot_p = (b - 1) % NBUF
            if b == 0:
                @pl.when(p < NG)
                def _refill0():
                    @pl.when(i >= 1)
                    def _drain_prev():
                        store_desc(h - 1, slot_p).wait()
                    start_gathers(p, slot_p)
            else:
                @pl.when(p < NG)
                def _refill():
                    store_desc(h - 1, slot_p).wait()
                    start_gathers(p, slot_p)

    for b in range(NBUF):
        store_desc(NG - NBUF + b, b).wait()


_emb_lookup = functools.partial(
    pl.kernel,
    out_type=jax.ShapeDtypeStruct((SEQ, BATCH, DIM), jnp.float32),
    mesh=plsc.VectorSubcoreMesh(
        core_axis_name="c", subcore_axis_name="s",
        num_cores=NC, num_subcores=NS),
    scratch_types=[
        pltpu.VMEM((NG, G), jnp.int32),
        pltpu.VMEM((NBUF, G, DIM), jnp.float32),
        pltpu.SMEM((NBUF, G), jnp.int32),
        pltpu.SemaphoreType.DMA((NBUF,)),
        pltpu.SemaphoreType.DMA((NBUF,)),
    ],
    compiler_params=pltpu.CompilerParams(
        needs_layout_passes=False, use_tc_tiling_on_sc=True),
)(_emb_body)


@jax.jit
def kernel(src_input, table):
    idx = src_input.reshape(N).astype(jnp.int32).reshape(NW, NG, G)
    return _emb_lookup(table, idx)
